# Initial kernel scaffold; baseline (speedup 1.0000x reference)
import jax
import jax.numpy as jnp
from jax.experimental import pallas as pl


def kernel(x, batch):
    raise NotImplementedError("write your pallas kernel here")


# speedup vs baseline: 3.6347x; 3.6347x over previous
import functools

import jax
import jax.numpy as jnp
from jax import lax
from jax.experimental import pallas as pl
from jax.experimental.pallas import tpu as pltpu
from jax.experimental.pallas import tpu_sc as plsc

N = 100000
D = 128
G = 64
NW = 32
RPW = N // NW
B = 125
NB = RPW // B
RPW_PAD = 3136
NV = RPW_PAD // 16

_mesh = plsc.VectorSubcoreMesh(core_axis_name="c", subcore_axis_name="s")


@functools.partial(
    pl.kernel,
    out_type=(
        jax.ShapeDtypeStruct((2, G, D), jnp.float32),
        jax.ShapeDtypeStruct((NW, G, 16), jnp.float32),
    ),
    mesh=_mesh,
    scratch_types=[
        pltpu.VMEM((B, D), jnp.float32),
        pltpu.VMEM((NB, B), jnp.int32),
        pltpu.VMEM((RPW_PAD,), jnp.int32),
        pltpu.VMEM((G, 16), jnp.float32),
        pltpu.VMEM_SHARED((G, D), jnp.float32),
    ],
)
def _seg_sum_sc(x_hbm, idx_hbm, idxf_hbm, sums_hbm, cnts_hbm, xbuf, idxbuf,
                idxflat, cntf, acc_sp):
    c = lax.axis_index("c")
    s = lax.axis_index("s")
    wid = c * 16 + s
    zeros16 = jnp.zeros((16,), jnp.float32)
    rows_per_tile = G // 16

    for i in range(rows_per_tile):
        for g in range(D // 16):
            xbuf[i, pl.ds(g * 16, 16)] = zeros16
    pltpu.sync_copy(xbuf.at[pl.ds(0, rows_per_tile)],
                    acc_sp.at[pl.ds(s * rows_per_tile, rows_per_tile)])

    pltpu.sync_copy(idx_hbm.at[wid], idxbuf)
    pltpu.sync_copy(idxf_hbm.at[wid], idxflat)

    for k in range(G):
        cntf[k, :] = zeros16

    def hist_body(g, carry):
        gs = jnp.full((16,), g, jnp.int32)
        acc = jnp.zeros((16,), jnp.float32)
        one = jnp.ones((16,), jnp.float32)
        for j in range(NV):
            v = idxflat[pl.ds(j * 16, 16)]
            acc = acc + jnp.where(v == gs, one, zeros16)
        cntf[g, :] = acc
        return carry

    lax.fori_loop(0, G, hist_body, 0)
    pltpu.sync_copy(cntf, cnts_hbm.at[wid])
    plsc.subcore_barrier()

    def sync_body(b, carry):
        pltpu.sync_copy(x_hbm.at[wid, b], xbuf)
        pltpu.sync_copy(xbuf, acc_sp.at[idxbuf.at[b]], add=True)
        return carry

    lax.fori_loop(0, NB, sync_body, 0)
    plsc.subcore_barrier()

    pltpu.sync_copy(acc_sp.at[pl.ds(s * rows_per_tile, rows_per_tile)],
                    xbuf.at[pl.ds(0, rows_per_tile)])
    pltpu.sync_copy(xbuf.at[pl.ds(0, rows_per_tile)],
                    sums_hbm.at[c, pl.ds(s * rows_per_tile, rows_per_tile)])


def _combine_tc(sums_ref, cnts_ref, out_ref):
    total = sums_ref[0] + sums_ref[1]
    cnt = jnp.sum(cnts_ref[...], axis=(0, 2))
    out_ref[...] = total / cnt[:, None]


def kernel(x, batch):
    seg = batch.astype(jnp.int32)
    xr = x.reshape(NW, NB, B, D)
    segr = seg.reshape(NW, NB, B)
    segf = jnp.pad(seg.reshape(NW, RPW), ((0, 0), (0, RPW_PAD - RPW)),
                   constant_values=127)
    sums, cnts = _seg_sum_sc(xr, segr, segf)
    return pl.pallas_call(
        _combine_tc,
        out_shape=jax.ShapeDtypeStruct((G, D), jnp.float32),
    )(sums, cnts)


# speedup vs baseline: 4.0673x; 1.1190x over previous
import functools

import jax
import jax.numpy as jnp
from jax import lax
from jax.experimental import pallas as pl
from jax.experimental.pallas import tpu as pltpu
from jax.experimental.pallas import tpu_sc as plsc

N = 100000
D = 128
G = 64
NW = 32
RPW = N // NW
B = 125
NB = RPW // B
RPW_PAD = 3136
NV = RPW_PAD // 16

_mesh = plsc.VectorSubcoreMesh(core_axis_name="c", subcore_axis_name="s")


@functools.partial(
    pl.kernel,
    out_type=(
        jax.ShapeDtypeStruct((2, G, D), jnp.float32),
        jax.ShapeDtypeStruct((NW, G, 16), jnp.float32),
    ),
    mesh=_mesh,
    scratch_types=[
        pltpu.VMEM((2, B, D), jnp.float32),
        pltpu.SemaphoreType.DMA,
        pltpu.VMEM((NB, B), jnp.int32),
        pltpu.VMEM((RPW_PAD,), jnp.int32),
        pltpu.VMEM((G, 16), jnp.float32),
        pltpu.VMEM_SHARED((G, D), jnp.float32),
    ],
)
def _seg_sum_sc(x_hbm, idx_hbm, idxf_hbm, sums_hbm, cnts_hbm, xbuf2, gsem,
                idxbuf, idxflat, cntf, acc_sp):
    xbuf = xbuf2.at[0]
    c = lax.axis_index("c")
    s = lax.axis_index("s")
    wid = c * 16 + s
    zeros16 = jnp.zeros((16,), jnp.float32)
    rows_per_tile = G // 16

    for i in range(rows_per_tile):
        for g in range(D // 16):
            xbuf[i, pl.ds(g * 16, 16)] = zeros16
    pltpu.sync_copy(xbuf.at[pl.ds(0, rows_per_tile)],
                    acc_sp.at[pl.ds(s * rows_per_tile, rows_per_tile)])

    pltpu.sync_copy(idx_hbm.at[wid], idxbuf)
    pltpu.sync_copy(idxf_hbm.at[wid], idxflat)

    for k in range(G):
        cntf[k, :] = zeros16

    def hist_body(g, carry):
        gs = jnp.full((16,), g, jnp.int32)
        acc = jnp.zeros((16,), jnp.float32)
        one = jnp.ones((16,), jnp.float32)
        for j in range(NV):
            v = idxflat[pl.ds(j * 16, 16)]
            acc = acc + jnp.where(v == gs, one, zeros16)
        cntf[g, :] = acc
        return carry

    lax.fori_loop(0, G, hist_body, 0)
    pltpu.sync_copy(cntf, cnts_hbm.at[wid])
    plsc.subcore_barrier()

    pltpu.async_copy(x_hbm.at[wid, 0], xbuf2.at[0], gsem)

    def sync_body(b, carry):
        buf = xbuf2.at[b % 2]
        pltpu.make_async_copy(x_hbm.at[wid, b], buf, gsem).wait()

        @pl.when(b + 1 < NB)
        def _():
            pltpu.async_copy(x_hbm.at[wid, b + 1], xbuf2.at[(b + 1) % 2], gsem)

        pltpu.sync_copy(buf, acc_sp.at[idxbuf.at[b]], add=True)
        return carry

    lax.fori_loop(0, NB, sync_body, 0)
    plsc.subcore_barrier()

    pltpu.sync_copy(acc_sp.at[pl.ds(s * rows_per_tile, rows_per_tile)],
                    xbuf.at[pl.ds(0, rows_per_tile)])
    pltpu.sync_copy(xbuf.at[pl.ds(0, rows_per_tile)],
                    sums_hbm.at[c, pl.ds(s * rows_per_tile, rows_per_tile)])


def _combine_tc(sums_ref, cnts_ref, out_ref):
    total = sums_ref[0] + sums_ref[1]
    cnt = jnp.sum(cnts_ref[...], axis=(0, 2))
    out_ref[...] = total / cnt[:, None]


def kernel(x, batch):
    seg = batch.astype(jnp.int32)
    xr = x.reshape(NW, NB, B, D)
    segr = seg.reshape(NW, NB, B)
    segf = jnp.pad(seg.reshape(NW, RPW), ((0, 0), (0, RPW_PAD - RPW)),
                   constant_values=127)
    sums, cnts = _seg_sum_sc(xr, segr, segf)
    return pl.pallas_call(
        _combine_tc,
        out_shape=jax.ShapeDtypeStruct((G, D), jnp.float32),
    )(sums, cnts)


# speedup vs baseline: 5.9200x; 1.4555x over previous
import functools

import jax
import jax.numpy as jnp
from jax import lax
from jax.experimental import pallas as pl
from jax.experimental.pallas import tpu as pltpu
from jax.experimental.pallas import tpu_sc as plsc

N = 100000
D = 128
G = 64
NW = 32
B = 120
NB = 26
RPW = B * NB
TAIL = N - NW * RPW
TB = 32
NTW = TAIL // TB
HRW = N // NW
HRW_PAD = 3136
NV = HRW_PAD // 16

_mesh = plsc.VectorSubcoreMesh(core_axis_name="c", subcore_axis_name="s")


@functools.partial(
    pl.kernel,
    out_type=(
        jax.ShapeDtypeStruct((2, G, D), jnp.float32),
        jax.ShapeDtypeStruct((NW, G, 16), jnp.float32),
    ),
    mesh=_mesh,
    scratch_types=[
        pltpu.VMEM((2, B, D), jnp.float32),
        pltpu.VMEM((TB, D), jnp.float32),
        pltpu.SemaphoreType.DMA,
        pltpu.VMEM((NB, B), jnp.int32),
        pltpu.VMEM((1, TB), jnp.int32),
        pltpu.VMEM((HRW_PAD,), jnp.int32),
        pltpu.VMEM((G, 16), jnp.float32),
        pltpu.VMEM_SHARED((G, D), jnp.float32),
    ],
)
def _seg_sum_sc(x_hbm, idx_hbm, idxt_hbm, idxf_hbm, sums_hbm, cnts_hbm,
                xbuf2, xtail, gsem, idxbuf, idxtail, idxflat, cntf, acc_sp):
    xbuf = xbuf2.at[0]
    c = lax.axis_index("c")
    s = lax.axis_index("s")
    wid = c * 16 + s
    zeros16 = jnp.zeros((16,), jnp.float32)
    rows_per_tile = G // 16

    for i in range(rows_per_tile):
        for g in range(D // 16):
            xbuf[i, pl.ds(g * 16, 16)] = zeros16
    pltpu.sync_copy(xbuf.at[pl.ds(0, rows_per_tile)],
                    acc_sp.at[pl.ds(s * rows_per_tile, rows_per_tile)])

    pltpu.sync_copy(idx_hbm.at[wid], idxbuf)
    pltpu.sync_copy(idxf_hbm.at[wid], idxflat)

    for k in range(G):
        cntf[k, :] = zeros16

    def hist_body(g, carry):
        gs = jnp.full((16,), g, jnp.int32)
        acc = jnp.zeros((16,), jnp.float32)
        one = jnp.ones((16,), jnp.float32)
        for j in range(NV):
            v = idxflat[pl.ds(j * 16, 16)]
            acc = acc + jnp.where(v == gs, one, zeros16)
        cntf[g, :] = acc
        return carry

    lax.fori_loop(0, G, hist_body, 0)
    pltpu.sync_copy(cntf, cnts_hbm.at[wid])
    plsc.subcore_barrier()

    base = wid * RPW
    pltpu.async_copy(x_hbm.at[pl.ds(base, B)], xbuf2.at[0], gsem)

    def sync_body(b, carry):
        buf = xbuf2.at[b % 2]
        pltpu.make_async_copy(x_hbm.at[pl.ds(base + b * B, B)], buf, gsem).wait()

        @pl.when(b + 1 < NB)
        def _():
            pltpu.async_copy(x_hbm.at[pl.ds(base + (b + 1) * B, B)],
                             xbuf2.at[(b + 1) % 2], gsem)

        pltpu.sync_copy(buf, acc_sp.at[idxbuf.at[b]], add=True)
        return carry

    lax.fori_loop(0, NB, sync_body, 0)

    @pl.when(wid < NTW)
    def _():
        pltpu.sync_copy(idxt_hbm.at[wid], idxtail)
        pltpu.sync_copy(x_hbm.at[pl.ds(NW * RPW + wid * TB, TB)], xtail)
        pltpu.sync_copy(xtail, acc_sp.at[idxtail.at[0]], add=True)

    plsc.subcore_barrier()

    pltpu.sync_copy(acc_sp.at[pl.ds(s * rows_per_tile, rows_per_tile)],
                    xbuf.at[pl.ds(0, rows_per_tile)])
    pltpu.sync_copy(xbuf.at[pl.ds(0, rows_per_tile)],
                    sums_hbm.at[c, pl.ds(s * rows_per_tile, rows_per_tile)])


def _combine_tc(sums_ref, cnts_ref, out_ref):
    total = sums_ref[0] + sums_ref[1]
    cnt = jnp.sum(cnts_ref[...], axis=(0, 2))
    out_ref[...] = total / cnt[:, None]


def kernel(x, batch):
    seg = batch.astype(jnp.int32)
    segr = seg[:NW * RPW].reshape(NW, NB, B)
    segt = seg[NW * RPW:].reshape(NTW, 1, TB)
    segf = jnp.pad(seg.reshape(NW, HRW), ((0, 0), (0, HRW_PAD - HRW)),
                   constant_values=127)
    sums, cnts = _seg_sum_sc(x, segr, segt, segf)
    return pl.pallas_call(
        _combine_tc,
        out_shape=jax.ShapeDtypeStruct((G, D), jnp.float32),
    )(sums, cnts)


# speedup vs baseline: 5.9260x; 1.0010x over previous
import functools

import jax
import jax.numpy as jnp
from jax import lax
from jax.experimental import pallas as pl
from jax.experimental.pallas import tpu as pltpu
from jax.experimental.pallas import tpu_sc as plsc

N = 100000
D = 128
G = 64
NW = 32
B = 120
NB = 26
RPW = B * NB
TAIL = N - NW * RPW
TB = 32
NTW = TAIL // TB
HRW = N // NW
HRW_PAD = 3136
NV = HRW_PAD // 16

_mesh = plsc.VectorSubcoreMesh(core_axis_name="c", subcore_axis_name="s")


@functools.partial(
    pl.kernel,
    out_type=(
        jax.ShapeDtypeStruct((2, G, D), jnp.float32),
        jax.ShapeDtypeStruct((NW, G, 16), jnp.float32),
    ),
    mesh=_mesh,
    scratch_types=[
        pltpu.VMEM((2, B, D), jnp.float32),
        pltpu.VMEM((TB, D), jnp.float32),
        pltpu.SemaphoreType.DMA,
        pltpu.SemaphoreType.DMA,
        pltpu.VMEM((NB, B), jnp.int32),
        pltpu.VMEM((1, TB), jnp.int32),
        pltpu.VMEM((HRW_PAD,), jnp.int32),
        pltpu.VMEM((G, 16), jnp.float32),
        pltpu.VMEM_SHARED((G, D), jnp.float32),
    ],
)
def _seg_sum_sc(x_hbm, idx_hbm, idxt_hbm, idxf_hbm, sums_hbm, cnts_hbm,
                xbuf2, xtail, gsem, ssem, idxbuf, idxtail, idxflat, cntf,
                acc_sp):
    xbuf = xbuf2.at[0]
    c = lax.axis_index("c")
    s = lax.axis_index("s")
    wid = c * 16 + s
    zeros16 = jnp.zeros((16,), jnp.float32)
    rows_per_tile = G // 16

    for i in range(rows_per_tile):
        for g in range(D // 16):
            xbuf[i, pl.ds(g * 16, 16)] = zeros16
    pltpu.sync_copy(xbuf.at[pl.ds(0, rows_per_tile)],
                    acc_sp.at[pl.ds(s * rows_per_tile, rows_per_tile)])

    pltpu.sync_copy(idx_hbm.at[wid], idxbuf)
    pltpu.sync_copy(idxf_hbm.at[wid], idxflat)

    for k in range(G):
        cntf[k, :] = zeros16

    def hist_body(g, carry):
        gs = jnp.full((16,), g, jnp.int32)
        acc = jnp.zeros((16,), jnp.float32)
        one = jnp.ones((16,), jnp.float32)
        for j in range(NV):
            v = idxflat[pl.ds(j * 16, 16)]
            acc = acc + jnp.where(v == gs, one, zeros16)
        cntf[g, :] = acc
        return carry

    lax.fori_loop(0, G, hist_body, 0)
    pltpu.sync_copy(cntf, cnts_hbm.at[wid])
    plsc.subcore_barrier()

    base = wid * RPW
    pltpu.async_copy(x_hbm.at[pl.ds(base, B)], xbuf2.at[0], gsem)

    def sync_body(b, carry):
        buf = xbuf2.at[b % 2]
        pltpu.make_async_copy(x_hbm.at[pl.ds(base + b * B, B)], buf, gsem).wait()
        pltpu.async_copy(buf, acc_sp.at[idxbuf.at[b]], ssem, add=True)

        @pl.when(b >= 1)
        def _():
            pltpu.make_async_copy(xbuf2.at[(b + 1) % 2],
                                  acc_sp.at[idxbuf.at[b - 1]], ssem).wait()

        @pl.when(b + 1 < NB)
        def _():
            pltpu.async_copy(x_hbm.at[pl.ds(base + (b + 1) * B, B)],
                             xbuf2.at[(b + 1) % 2], gsem)

        return carry

    lax.fori_loop(0, NB, sync_body, 0)
    pltpu.make_async_copy(xbuf2.at[(NB - 1) % 2],
                          acc_sp.at[idxbuf.at[NB - 1]], ssem).wait()

    @pl.when(wid < NTW)
    def _():
        pltpu.sync_copy(idxt_hbm.at[wid], idxtail)
        pltpu.sync_copy(x_hbm.at[pl.ds(NW * RPW + wid * TB, TB)], xtail)
        pltpu.sync_copy(xtail, acc_sp.at[idxtail.at[0]], add=True)

    plsc.subcore_barrier()

    pltpu.sync_copy(acc_sp.at[pl.ds(s * rows_per_tile, rows_per_tile)],
                    xbuf.at[pl.ds(0, rows_per_tile)])
    pltpu.sync_copy(xbuf.at[pl.ds(0, rows_per_tile)],
                    sums_hbm.at[c, pl.ds(s * rows_per_tile, rows_per_tile)])


def _combine_tc(sums_ref, cnts_ref, out_ref):
    total = sums_ref[0] + sums_ref[1]
    cnt = jnp.sum(cnts_ref[...], axis=(0, 2))
    out_ref[...] = total / cnt[:, None]


def kernel(x, batch):
    seg = batch.astype(jnp.int32)
    segr = seg[:NW * RPW].reshape(NW, NB, B)
    segt = seg[NW * RPW:].reshape(NTW, 1, TB)
    segf = jnp.pad(seg.reshape(NW, HRW), ((0, 0), (0, HRW_PAD - HRW)),
                   constant_values=127)
    sums, cnts = _seg_sum_sc(x, segr, segt, segf)
    return pl.pallas_call(
        _combine_tc,
        out_shape=jax.ShapeDtypeStruct((G, D), jnp.float32),
    )(sums, cnts)
